# Initial kernel scaffold; baseline (speedup 1.0000x reference)
import jax
import jax.numpy as jnp
from jax.experimental import pallas as pl


def kernel(user_indices, item_indices, ue_gmf, ie_gmf, ue_mlp, ie_mlp, W1, b1, g1, be1, W2, b2, g2, be2, W3, b3, g3, be3, Wo, bo):
    raise NotImplementedError("write your pallas kernel here")


# speedup vs baseline: 2.0260x; 2.0260x over previous
import functools

import jax
import jax.numpy as jnp
from jax import lax
from jax.experimental import pallas as pl
from jax.experimental.pallas import tpu as pltpu
from jax.experimental.pallas import tpu_sc as plsc

B = 16384
D = 128
EPS = 1e-5


try:
    _info = plsc.get_sparse_core_info()
    _NC = _info.num_cores
    _NS = _info.num_subcores
except Exception:
    _NC, _NS = 2, 16
_NW = _NC * _NS
_BPW = B // _NW
_CH = 128
_NCH = _BPW // _CH


def _sc_gather4(uidx, iidx, ue_gmf, ie_gmf, ue_mlp, ie_mlp):
    mesh = plsc.VectorSubcoreMesh(core_axis_name="c", subcore_axis_name="s")
    f32 = jnp.float32

    @functools.partial(
        pl.kernel,
        mesh=mesh,
        out_type=[jax.ShapeDtypeStruct((B, D), f32) for _ in range(4)],
        scratch_types=[
            pltpu.VMEM((_CH,), jnp.int32),
            pltpu.VMEM((_CH,), jnp.int32),
            pltpu.VMEM((_CH, D), f32),
            pltpu.VMEM((_CH, D), f32),
            pltpu.VMEM((_CH, D), f32),
            pltpu.VMEM((_CH, D), f32),
            pltpu.SemaphoreType.DMA,
        ],
    )
    def gather_k(uidx_h, iidx_h, ug_t, ig_t, um_t, im_t,
                 ug_o, ig_o, um_o, im_o,
                 uv, iv, b0, b1, b2, b3, sem):
        wid = lax.axis_index("s") * _NC + lax.axis_index("c")
        base = wid * _BPW
        for c in range(_NCH):
            off = base + c * _CH
            pltpu.sync_copy(uidx_h.at[pl.ds(off, _CH)], uv)
            pltpu.sync_copy(iidx_h.at[pl.ds(off, _CH)], iv)
            cp0 = pltpu.async_copy(ug_t.at[uv], b0, sem)
            cp1 = pltpu.async_copy(ig_t.at[iv], b1, sem)
            cp2 = pltpu.async_copy(um_t.at[uv], b2, sem)
            cp3 = pltpu.async_copy(im_t.at[iv], b3, sem)
            cp0.wait()
            cp1.wait()
            cp2.wait()
            cp3.wait()
            pltpu.sync_copy(b0, ug_o.at[pl.ds(off, _CH)])
            pltpu.sync_copy(b1, ig_o.at[pl.ds(off, _CH)])
            pltpu.sync_copy(b2, um_o.at[pl.ds(off, _CH)])
            pltpu.sync_copy(b3, im_o.at[pl.ds(off, _CH)])

    return gather_k(uidx, iidx, ue_gmf, ie_gmf, ue_mlp, ie_mlp)


_BLK = 2048
_NB = B // _BLK


def _leaky(z):
    return jnp.where(z > 0, z, 0.1 * z)


def _accum_stats(a, st_ref):
    ps = jnp.stack([jnp.sum(a, axis=0), jnp.sum(a * a, axis=0)])

    @pl.when(pl.program_id(0) == 0)
    def _():
        st_ref[...] = ps

    @pl.when(pl.program_id(0) > 0)
    def _():
        st_ref[...] = st_ref[...] + ps


def _norm_params(st, g, be):
    m = st[0] * (1.0 / B)
    var = st[1] * (1.0 / B) - m * m
    scale = g * lax.rsqrt(var + EPS)
    shift = be - m * scale
    return scale, shift


def _stage1_body(um_ref, im_ref, w_ref, b_ref, h_ref, st_ref):
    w = w_ref[...]
    z = (jnp.dot(um_ref[...], w[:D], preferred_element_type=jnp.float32)
         + jnp.dot(im_ref[...], w[D:], preferred_element_type=jnp.float32)
         + b_ref[...])
    a = _leaky(z)
    h_ref[...] = a
    _accum_stats(a, st_ref)


def _stageN_body(h_in_ref, st_in_ref, g_ref, be_ref, w_ref, b_ref,
                 h_ref, st_ref):
    scale, shift = _norm_params(st_in_ref[...], g_ref[...], be_ref[...])
    x = h_in_ref[...] * scale + shift
    z = jnp.dot(x, w_ref[...], preferred_element_type=jnp.float32) + b_ref[...]
    a = _leaky(z)
    h_ref[...] = a
    _accum_stats(a, st_ref)


def _final_body(h3_ref, st_ref, g_ref, be_ref, ug_ref, ig_ref,
                wo_ref, bo_ref, o_ref):
    scale, shift = _norm_params(st_ref[...], g_ref[...], be_ref[...])
    z3 = h3_ref[...] * scale + shift
    gmf = ug_ref[...] * ig_ref[...]
    wo = wo_ref[...][:, 0]
    s = (jnp.sum(gmf * wo[:D] + z3 * wo[D:], axis=1) + bo_ref[0])
    o_ref[...] = jax.nn.sigmoid(s)


def _full_spec(ndim):
    return pl.BlockSpec(None, lambda i: (0,) * ndim)


def _row_spec(h):
    return pl.BlockSpec((_BLK, h), lambda i: (i, 0))


def kernel(user_indices, item_indices, ue_gmf, ie_gmf, ue_mlp, ie_mlp,
           W1, b1, g1, be1, W2, b2, g2, be2, W3, b3, g3, be3, Wo, bo):
    uidx = user_indices.astype(jnp.int32)
    iidx = item_indices.astype(jnp.int32)

    ug, ig, um, im = _sc_gather4(uidx, iidx, ue_gmf, ie_gmf, ue_mlp, ie_mlp)

    f32 = jnp.float32
    h1, st1 = pl.pallas_call(
        _stage1_body,
        grid=(_NB,),
        in_specs=[_row_spec(D), _row_spec(D), _full_spec(2), _full_spec(1)],
        out_specs=[_row_spec(512), _full_spec(2)],
        out_shape=[jax.ShapeDtypeStruct((B, 512), f32),
                   jax.ShapeDtypeStruct((2, 512), f32)],
    )(um, im, W1, b1)

    h2, st2 = pl.pallas_call(
        _stageN_body,
        grid=(_NB,),
        in_specs=[_row_spec(512), _full_spec(2), _full_spec(1), _full_spec(1),
                  _full_spec(2), _full_spec(1)],
        out_specs=[_row_spec(256), _full_spec(2)],
        out_shape=[jax.ShapeDtypeStruct((B, 256), f32),
                   jax.ShapeDtypeStruct((2, 256), f32)],
    )(h1, st1, g1, be1, W2, b2)

    h3, st3 = pl.pallas_call(
        _stageN_body,
        grid=(_NB,),
        in_specs=[_row_spec(256), _full_spec(2), _full_spec(1), _full_spec(1),
                  _full_spec(2), _full_spec(1)],
        out_specs=[_row_spec(128), _full_spec(2)],
        out_shape=[jax.ShapeDtypeStruct((B, 128), f32),
                   jax.ShapeDtypeStruct((2, 128), f32)],
    )(h2, st2, g2, be2, W3, b3)

    out = pl.pallas_call(
        _final_body,
        grid=(_NB,),
        in_specs=[_row_spec(128), _full_spec(2), _full_spec(1), _full_spec(1),
                  _row_spec(D), _row_spec(D), _full_spec(2), _full_spec(1)],
        out_specs=pl.BlockSpec((_BLK,), lambda i: (i,)),
        out_shape=jax.ShapeDtypeStruct((B,), f32),
    )(h3, st3, g3, be3, ug, ig, Wo, bo)

    return out


# speedup vs baseline: 2.1705x; 1.0713x over previous
import functools

import jax
import jax.numpy as jnp
from jax import lax
from jax.experimental import pallas as pl
from jax.experimental.pallas import tpu as pltpu
from jax.experimental.pallas import tpu_sc as plsc

B = 16384
D = 128
EPS = 1e-5


try:
    _info = plsc.get_sparse_core_info()
    _NC = _info.num_cores
    _NS = _info.num_subcores
except Exception:
    _NC, _NS = 2, 16
_NW = _NC * _NS
_BPW = B // _NW
_CH = 128
_NCH = _BPW // _CH


def _sc_gather2(uidx, iidx, tab_u, tab_i):
    mesh = plsc.VectorSubcoreMesh(core_axis_name="c", subcore_axis_name="s")
    f32 = jnp.float32

    @functools.partial(
        pl.kernel,
        mesh=mesh,
        out_type=[jax.ShapeDtypeStruct((B, D), f32) for _ in range(2)],
        scratch_types=[
            pltpu.VMEM((_CH,), jnp.int32),
            pltpu.VMEM((_CH,), jnp.int32),
            pltpu.VMEM((_CH,), jnp.int32),
            pltpu.VMEM((_CH,), jnp.int32),
            pltpu.VMEM((_CH, D), f32),
            pltpu.VMEM((_CH, D), f32),
            pltpu.VMEM((_CH, D), f32),
            pltpu.VMEM((_CH, D), f32),
            pltpu.SemaphoreType.DMA,
            pltpu.SemaphoreType.DMA,
            pltpu.SemaphoreType.DMA,
            pltpu.SemaphoreType.DMA,
        ],
    )
    def gather_k(uidx_h, iidx_h, tu_h, ti_h, uo_h, io_h,
                 uv0, uv1, iv0, iv1, bu0, bu1, bi0, bi1,
                 g0, g1, s0, s1):
        uv = (uv0, uv1)
        iv = (iv0, iv1)
        bu = (bu0, bu1)
        bi = (bi0, bi1)
        gsem = (g0, g1)
        ssem = (s0, s1)
        wid = lax.axis_index("s") * _NC + lax.axis_index("c")
        base = wid * _BPW

        gh = [None] * _NCH
        sh = [None] * _NCH
        pltpu.sync_copy(uidx_h.at[pl.ds(base, _CH)], uv[0])
        pltpu.sync_copy(iidx_h.at[pl.ds(base, _CH)], iv[0])
        for c in range(_NCH):
            p = c % 2
            if c >= 2:
                sh[c - 2][0].wait()
                sh[c - 2][1].wait()
            gh[c] = (pltpu.async_copy(tu_h.at[uv[p]], bu[p], gsem[p]),
                     pltpu.async_copy(ti_h.at[iv[p]], bi[p], gsem[p]))
            if c + 1 < _NCH:
                off_n = base + (c + 1) * _CH
                pltpu.sync_copy(uidx_h.at[pl.ds(off_n, _CH)], uv[1 - p])
                pltpu.sync_copy(iidx_h.at[pl.ds(off_n, _CH)], iv[1 - p])
            if c >= 1:
                q = 1 - p
                off_p = base + (c - 1) * _CH
                gh[c - 1][0].wait()
                gh[c - 1][1].wait()
                sh[c - 1] = (
                    pltpu.async_copy(bu[q], uo_h.at[pl.ds(off_p, _CH)],
                                     ssem[q]),
                    pltpu.async_copy(bi[q], io_h.at[pl.ds(off_p, _CH)],
                                     ssem[q]),
                )
        c = _NCH - 1
        p = c % 2
        gh[c][0].wait()
        gh[c][1].wait()
        off_p = base + c * _CH
        sh[c] = (pltpu.async_copy(bu[p], uo_h.at[pl.ds(off_p, _CH)], ssem[p]),
                 pltpu.async_copy(bi[p], io_h.at[pl.ds(off_p, _CH)], ssem[p]))
        sh[c - 1][0].wait()
        sh[c - 1][1].wait()
        sh[c][0].wait()
        sh[c][1].wait()

    return gather_k(uidx, iidx, tab_u, tab_i)


_BLK = 2048
_NB = B // _BLK


def _leaky(z):
    return jnp.where(z > 0, z, 0.1 * z)


def _accum_stats(a, st_ref):
    ps = jnp.stack([jnp.sum(a, axis=0), jnp.sum(a * a, axis=0)])

    @pl.when(pl.program_id(0) == 0)
    def _():
        st_ref[...] = ps

    @pl.when(pl.program_id(0) > 0)
    def _():
        st_ref[...] = st_ref[...] + ps


def _norm_params(st, g, be):
    m = st[0] * (1.0 / B)
    var = st[1] * (1.0 / B) - m * m
    scale = g * lax.rsqrt(var + EPS)
    shift = be - m * scale
    return scale, shift


def _stage1_body(um_ref, im_ref, w_ref, b_ref, h_ref, st_ref):
    w = w_ref[...]
    z = (jnp.dot(um_ref[...], w[:D], preferred_element_type=jnp.float32)
         + jnp.dot(im_ref[...], w[D:], preferred_element_type=jnp.float32)
         + b_ref[...])
    a = _leaky(z)
    h_ref[...] = a
    _accum_stats(a, st_ref)


def _stageN_body(h_in_ref, st_in_ref, g_ref, be_ref, w_ref, b_ref,
                 h_ref, st_ref):
    scale, shift = _norm_params(st_in_ref[...], g_ref[...], be_ref[...])
    x = h_in_ref[...] * scale + shift
    z = jnp.dot(x, w_ref[...], preferred_element_type=jnp.float32) + b_ref[...]
    a = _leaky(z)
    h_ref[...] = a
    _accum_stats(a, st_ref)


def _final_body(h3_ref, st_ref, g_ref, be_ref, ug_ref, ig_ref,
                wo_ref, bo_ref, o_ref):
    scale, shift = _norm_params(st_ref[...], g_ref[...], be_ref[...])
    z3 = h3_ref[...] * scale + shift
    gmf = ug_ref[...] * ig_ref[...]
    wo = wo_ref[...][:, 0]
    s = (jnp.sum(gmf * wo[:D] + z3 * wo[D:], axis=1) + bo_ref[0])
    o_ref[...] = jax.nn.sigmoid(s)


def _full_spec(ndim):
    return pl.BlockSpec(None, lambda i: (0,) * ndim)


def _row_spec(h):
    return pl.BlockSpec((_BLK, h), lambda i: (i, 0))


def kernel(user_indices, item_indices, ue_gmf, ie_gmf, ue_mlp, ie_mlp,
           W1, b1, g1, be1, W2, b2, g2, be2, W3, b3, g3, be3, Wo, bo):
    uidx = user_indices.astype(jnp.int32)
    iidx = item_indices.astype(jnp.int32)

    um, im = _sc_gather2(uidx, iidx, ue_mlp, ie_mlp)
    ug, ig = _sc_gather2(uidx, iidx, ue_gmf, ie_gmf)

    f32 = jnp.float32
    h1, st1 = pl.pallas_call(
        _stage1_body,
        grid=(_NB,),
        in_specs=[_row_spec(D), _row_spec(D), _full_spec(2), _full_spec(1)],
        out_specs=[_row_spec(512), _full_spec(2)],
        out_shape=[jax.ShapeDtypeStruct((B, 512), f32),
                   jax.ShapeDtypeStruct((2, 512), f32)],
    )(um, im, W1, b1)

    h2, st2 = pl.pallas_call(
        _stageN_body,
        grid=(_NB,),
        in_specs=[_row_spec(512), _full_spec(2), _full_spec(1), _full_spec(1),
                  _full_spec(2), _full_spec(1)],
        out_specs=[_row_spec(256), _full_spec(2)],
        out_shape=[jax.ShapeDtypeStruct((B, 256), f32),
                   jax.ShapeDtypeStruct((2, 256), f32)],
    )(h1, st1, g1, be1, W2, b2)

    h3, st3 = pl.pallas_call(
        _stageN_body,
        grid=(_NB,),
        in_specs=[_row_spec(256), _full_spec(2), _full_spec(1), _full_spec(1),
                  _full_spec(2), _full_spec(1)],
        out_specs=[_row_spec(128), _full_spec(2)],
        out_shape=[jax.ShapeDtypeStruct((B, 128), f32),
                   jax.ShapeDtypeStruct((2, 128), f32)],
    )(h2, st2, g2, be2, W3, b3)

    out = pl.pallas_call(
        _final_body,
        grid=(_NB,),
        in_specs=[_row_spec(128), _full_spec(2), _full_spec(1), _full_spec(1),
                  _row_spec(D), _row_spec(D), _full_spec(2), _full_spec(1)],
        out_specs=pl.BlockSpec((_BLK,), lambda i: (i,)),
        out_shape=jax.ShapeDtypeStruct((B,), f32),
    )(h3, st3, g3, be3, ug, ig, Wo, bo)

    return out
